# baseline (speedup 1.0000x reference)
import functools

import jax
import jax.numpy as jnp
from jax import lax
from jax.experimental import pallas as pl
from jax.experimental.pallas import tpu as pltpu
from jax.experimental.pallas import tpu_sc as plsc

_NUM_ANNOTATOR = 1_000_000
_D = 64
_NC = 2
_NS = 16
_NW = _NC * _NS
_C = 128


def _sc_combined(questions, annotators, qtab, atab):
    n = questions.shape[0]
    per_w = n // _NW
    n_chunks = per_w // _C

    mesh = plsc.VectorSubcoreMesh(
        core_axis_name="c", subcore_axis_name="s",
        num_cores=_NC, num_subcores=_NS)

    @functools.partial(
        pl.kernel,
        mesh=mesh,
        compiler_params=pltpu.CompilerParams(use_tc_tiling_on_sc=False),
        out_type=jax.ShapeDtypeStruct((n, _D), jnp.float32),
        scratch_types=[
            pltpu.VMEM((_C,), jnp.int32),
            pltpu.VMEM((_C,), jnp.int32),
            pltpu.VMEM((_C, _D), jnp.float32),
            pltpu.VMEM((_C, _D), jnp.float32),
            pltpu.SemaphoreType.DMA,
            pltpu.SemaphoreType.DMA,
        ],
    )
    def sc_kernel(q_hbm, a_hbm, qtab_hbm, atab_hbm, out_hbm,
                  qidx, aidx, qrows, arows, sem_q, sem_a):
        wid = lax.axis_index("s") * _NC + lax.axis_index("c")
        base0 = wid * per_w

        def chunk_body(g, carry):
            base = base0 + g * _C
            pltpu.sync_copy(q_hbm.at[pl.ds(base, _C)], qidx)
            pltpu.sync_copy(a_hbm.at[pl.ds(base, _C)], aidx)

            def clamp_body(i, carry2):
                a = aidx[pl.ds(i * 16, 16)]
                aidx[pl.ds(i * 16, 16)] = jnp.where(
                    a < 0, jnp.full((16,), _NUM_ANNOTATOR, jnp.int32), a)
                return carry2
            lax.fori_loop(0, _C // 16, clamp_body, 0, unroll=True)

            cq = pltpu.async_copy(qtab_hbm.at[qidx], qrows, sem_q)
            ca = pltpu.async_copy(atab_hbm.at[aidx], arows, sem_a)
            cq.wait()
            ca.wait()

            def add_body(r, carry2):
                for k in range(_D // 16):
                    sl = pl.ds(k * 16, 16)
                    qrows[r, sl] = qrows[r, sl] + arows[r, sl]
                return carry2
            lax.fori_loop(0, _C, add_body, 0)

            pltpu.sync_copy(qrows, out_hbm.at[pl.ds(base, _C)])
            return carry

        lax.fori_loop(0, n_chunks, chunk_body, 0)

    return sc_kernel(questions, annotators, qtab, atab)


_TB = 64


def _tc_first(comb1, emb, xs):
    b, s = xs.shape[0], xs.shape[1]
    tb = _TB

    def body(comb_ref, emb_ref, xa_ref, xb_ref, feat_ref, par_ref):
        c3 = comb_ref[...].reshape(tb, s, _D)
        feat_ref[...] = jnp.concatenate(
            [c3, emb_ref[...], xa_ref[:, :, 1:]], axis=-1)
        par_ref[...] = xb_ref[:, :, 1:]

    return pl.pallas_call(
        body,
        grid=(b // 2 // tb,),
        in_specs=[
            pl.BlockSpec((tb * s, _D), lambda i: (i, 0)),
            pl.BlockSpec((tb, s, _D), lambda i: (i, 0, 0)),
            pl.BlockSpec((tb, s, 8), lambda i: (i, 0, 0)),
            pl.BlockSpec((2 * tb, s, 8), lambda i: (i, 0, 0)),
        ],
        out_specs=[
            pl.BlockSpec((tb, s, 135), lambda i: (i, 0, 0)),
            pl.BlockSpec((2 * tb, s, 7), lambda i: (i, 0, 0)),
        ],
        out_shape=[
            jax.ShapeDtypeStruct((b, s, 135), jnp.float32),
            jax.ShapeDtypeStruct((b, s, 7), jnp.float32),
        ],
    )(comb1, emb, xs, xs)


def _tc_second(comb2, emb, xs, feat):
    b, s = xs.shape[0], xs.shape[1]
    tb = _TB
    nb1 = b // 2 // tb

    def body(comb_ref, emb_ref, x_ref, alias_ref, feat_ref):
        c3 = comb_ref[...].reshape(tb, s, _D)
        feat_ref[...] = jnp.concatenate(
            [c3, emb_ref[...], x_ref[:, :, 1:]], axis=-1)

    return pl.pallas_call(
        body,
        grid=(nb1,),
        in_specs=[
            pl.BlockSpec((tb * s, _D), lambda i: (i, 0)),
            pl.BlockSpec((tb, s, _D), lambda i: (i + nb1, 0, 0)),
            pl.BlockSpec((tb, s, 8), lambda i: (i + nb1, 0, 0)),
            pl.BlockSpec((8, s, 135), lambda i: (0, 0, 0)),
        ],
        out_specs=pl.BlockSpec((tb, s, 135), lambda i: (i + nb1, 0, 0)),
        out_shape=jax.ShapeDtypeStruct((b, s, 135), jnp.float32),
        input_output_aliases={3: 0},
    )(comb2, emb, xs, feat)


def kernel(x, annotators, questions, embeddings, annotator_embedding,
           question_embedding):
    b, s = annotators.shape
    n = b * s
    h = n // 2
    q = questions.reshape(n).astype(jnp.int32)
    a = annotators.reshape(n).astype(jnp.int32)
    comb1 = _sc_combined(q[:h], a[:h], question_embedding, annotator_embedding)
    comb2 = _sc_combined(q[h:], a[h:], question_embedding, annotator_embedding)
    feat, par = _tc_first(comb1, embeddings, x)
    feat = _tc_second(comb2, embeddings, x, feat)
    return feat, par


# speedup vs baseline: 1.0189x; 1.0189x over previous
import functools

import jax
import jax.numpy as jnp
from jax import lax
from jax.experimental import pallas as pl
from jax.experimental.pallas import tpu as pltpu
from jax.experimental.pallas import tpu_sc as plsc

_NUM_ANNOTATOR = 1_000_000
_D = 64
_NC = 2
_NS = 16
_NW = _NC * _NS
_C = 128


def _sc_combined(questions, annotators, qtab, atab):
    n = questions.shape[0]
    per_w = n // _NW
    n_chunks = per_w // _C

    mesh = plsc.VectorSubcoreMesh(
        core_axis_name="c", subcore_axis_name="s",
        num_cores=_NC, num_subcores=_NS)

    @functools.partial(
        pl.kernel,
        mesh=mesh,
        compiler_params=pltpu.CompilerParams(use_tc_tiling_on_sc=False),
        out_type=jax.ShapeDtypeStruct((n, _D), jnp.float32),
        scratch_types=[
            pltpu.VMEM((per_w,), jnp.int32),
            pltpu.VMEM((per_w,), jnp.int32),
            pltpu.VMEM((_C, _D), jnp.float32),
            pltpu.VMEM((_C, _D), jnp.float32),
            pltpu.VMEM((_C, _D), jnp.float32),
            pltpu.VMEM((_C, _D), jnp.float32),
            pltpu.VMEM((_C, _D), jnp.float32),
            pltpu.VMEM((_C, _D), jnp.float32),
            pltpu.SemaphoreType.DMA,
            pltpu.SemaphoreType.DMA,
            pltpu.SemaphoreType.DMA,
            pltpu.SemaphoreType.DMA,
            pltpu.SemaphoreType.DMA,
            pltpu.SemaphoreType.DMA,
        ],
    )
    def sc_kernel(q_hbm, a_hbm, qtab_hbm, atab_hbm, out_hbm,
                  qidx, aidx, qr0, ar0, qr1, ar1, or0, or1,
                  sgq0, sga0, sgq1, sga1, so0, so1):
        wid = lax.axis_index("s") * _NC + lax.axis_index("c")
        base0 = wid * per_w

        pltpu.sync_copy(q_hbm.at[pl.ds(base0, per_w)], qidx)
        pltpu.sync_copy(a_hbm.at[pl.ds(base0, per_w)], aidx)

        def clamp_body(i, carry):
            a = aidx[pl.ds(i * 16, 16)]
            aidx[pl.ds(i * 16, 16)] = jnp.where(
                a < 0, jnp.full((16,), _NUM_ANNOTATOR, jnp.int32), a)
            return carry
        lax.fori_loop(0, per_w // 16, clamp_body, 0)

        def fire(g, qr, ar, sq, sa):
            isl = pl.ds(g * _C, _C)
            pltpu.async_copy(qtab_hbm.at[qidx.at[isl]], qr, sq)
            pltpu.async_copy(atab_hbm.at[aidx.at[isl]], ar, sa)

        def wait_gather(qr, ar, sq, sa):
            pltpu.make_async_copy(qtab_hbm.at[qidx.at[pl.ds(0, _C)]],
                                  qr, sq).wait()
            pltpu.make_async_copy(atab_hbm.at[aidx.at[pl.ds(0, _C)]],
                                  ar, sa).wait()

        def add(qr, ar, orr):
            def add_body(r, carry):
                for k in range(_D // 16):
                    sl = pl.ds(k * 16, 16)
                    orr[r, sl] = qr[r, sl] + ar[r, sl]
                return carry
            lax.fori_loop(0, _C, add_body, 0)

        def drain_store(orr, so, g):
            pltpu.make_async_copy(
                orr, out_hbm.at[pl.ds(base0 + g * _C, _C)], so).wait()

        fire(0, qr0, ar0, sgq0, sga0)

        def pair_body(p, carry):
            c0 = p * 2
            c1 = c0 + 1
            fire(c1, qr1, ar1, sgq1, sga1)
            wait_gather(qr0, ar0, sgq0, sga0)

            @pl.when(p > 0)
            def _():
                drain_store(or0, so0, 0)
            add(qr0, ar0, or0)

            @pl.when(c0 + 2 < n_chunks)
            def _():
                fire(c0 + 2, qr0, ar0, sgq0, sga0)
            pltpu.async_copy(or0, out_hbm.at[pl.ds(base0 + c0 * _C, _C)],
                             so0)

            wait_gather(qr1, ar1, sgq1, sga1)

            @pl.when(p > 0)
            def _():
                drain_store(or1, so1, 0)
            add(qr1, ar1, or1)
            pltpu.async_copy(or1, out_hbm.at[pl.ds(base0 + c1 * _C, _C)],
                             so1)
            return carry

        lax.fori_loop(0, n_chunks // 2, pair_body, 0)
        drain_store(or0, so0, 0)
        drain_store(or1, so1, 0)

    return sc_kernel(questions, annotators, qtab, atab)


def _tc_concat(comb, emb, xs):
    b, s = xs.shape[0], xs.shape[1]
    tb = 128

    def body(comb_ref, emb_ref, x_ref, feat_ref, par_ref):
        xt = x_ref[:, :, 1:]
        c3 = comb_ref[...].reshape(tb, s, _D)
        feat_ref[...] = jnp.concatenate([c3, emb_ref[...], xt], axis=-1)
        par_ref[...] = xt

    return pl.pallas_call(
        body,
        grid=(b // tb,),
        in_specs=[
            pl.BlockSpec((tb * s, _D), lambda i: (i, 0)),
            pl.BlockSpec((tb, s, _D), lambda i: (i, 0, 0)),
            pl.BlockSpec((tb, s, 8), lambda i: (i, 0, 0)),
        ],
        out_specs=[
            pl.BlockSpec((tb, s, 135), lambda i: (i, 0, 0)),
            pl.BlockSpec((tb, s, 7), lambda i: (i, 0, 0)),
        ],
        out_shape=[
            jax.ShapeDtypeStruct((b, s, 135), jnp.float32),
            jax.ShapeDtypeStruct((b, s, 7), jnp.float32),
        ],
    )(comb, emb, xs)


def kernel(x, annotators, questions, embeddings, annotator_embedding,
           question_embedding):
    b, s = annotators.shape
    n = b * s
    q = questions.reshape(n).astype(jnp.int32)
    a = annotators.reshape(n).astype(jnp.int32)
    comb = _sc_combined(q, a, question_embedding, annotator_embedding)
    return _tc_concat(comb, embeddings, x)
